# Initial kernel scaffold; baseline (speedup 1.0000x reference)
import jax
import jax.numpy as jnp
from jax.experimental import pallas as pl


def kernel(node_indices, adj_norm, embedding, W, b, alpha):
    raise NotImplementedError("write your pallas kernel here")


# speedup vs baseline: 1.0341x; 1.0341x over previous
import functools

import jax
import jax.numpy as jnp
from jax.experimental import pallas as pl
from jax.experimental.pallas import tpu as pltpu

N = 10000
D = 128
C = 2
L = 3
BM = 400


def _lightgcn_body(a_ref, x0_ref, wt_ref, b_ref, adj_ref, out_ref,
                   cur_ref, acc_ref):
    h = pl.program_id(0)
    i = pl.program_id(1)

    @pl.when((h == 0) & (i == 0))
    def _init():
        y0 = jnp.dot(x0_ref[...], wt_ref[...],
                     preferred_element_type=jnp.float32)
        cur_ref[0] = y0
        acc_ref[...] = a_ref[0] * y0

    p = h % 2
    new = jnp.dot(adj_ref[...], cur_ref[p],
                  preferred_element_type=jnp.float32)
    w = a_ref[h + 1]
    rows = pl.ds(i * BM, BM)
    acc_new = acc_ref[rows, :] + w * new
    acc_ref[rows, :] = acc_new
    cur_ref[1 - p, rows, :] = new

    @pl.when(h == L - 1)
    def _emit():
        out_ref[rows, :] = acc_new + b_ref[...]


def kernel(node_indices, adj_norm, embedding, W, b, alpha):
    a = jax.nn.softmax(alpha.astype(jnp.float32), axis=0)
    x0 = jnp.take(embedding, node_indices, axis=0)
    wt = W.T
    b2 = b.reshape(1, C)

    grid = (L, N // BM)
    out = pl.pallas_call(
        _lightgcn_body,
        grid=grid,
        in_specs=[
            pl.BlockSpec(memory_space=pltpu.SMEM),
            pl.BlockSpec((N, D), lambda h, i: (0, 0)),
            pl.BlockSpec((D, C), lambda h, i: (0, 0)),
            pl.BlockSpec((1, C), lambda h, i: (0, 0)),
            pl.BlockSpec((BM, N), lambda h, i: (i, 0)),
        ],
        out_specs=pl.BlockSpec((N, C), lambda h, i: (0, 0)),
        out_shape=jax.ShapeDtypeStruct((N, C), jnp.float32),
        scratch_shapes=[
            pltpu.VMEM((2, N, C), jnp.float32),
            pltpu.VMEM((N, C), jnp.float32),
        ],
        compiler_params=pltpu.CompilerParams(
            dimension_semantics=("arbitrary", "arbitrary"),
        ),
    )(a, x0, wt, b2, adj_norm)
    return out


# speedup vs baseline: 1.0532x; 1.0184x over previous
import jax
import jax.numpy as jnp
from jax.experimental import pallas as pl
from jax.experimental.pallas import tpu as pltpu

N = 10000
D = 128
C = 2
BM1 = 200
BM2 = 400


def _sweep1_body(x0_ref, wt_ref, adj_ref, y0_ref, y1_ref, a16_ref):
    i = pl.program_id(0)

    @pl.when(i == 0)
    def _init():
        y0_ref[...] = jnp.dot(x0_ref[...], wt_ref[...],
                              preferred_element_type=jnp.float32)

    a_blk = adj_ref[...]
    y1_ref[pl.ds(i * BM1, BM1), :] = jnp.dot(
        a_blk, y0_ref[...], preferred_element_type=jnp.float32)
    a16_ref[...] = a_blk.astype(jnp.bfloat16)


def _sweep23_body(a_ref, b_ref, y0_ref, y1_ref, a16_ref, out_ref,
                  acc_ref, y2_ref):
    h = pl.program_id(0)
    i = pl.program_id(1)

    @pl.when((h == 0) & (i == 0))
    def _init():
        acc_ref[...] = a_ref[0] * y0_ref[...] + a_ref[1] * y1_ref[...]

    rows = pl.ds(i * BM2, BM2)
    a16_blk = a16_ref[...]

    @pl.when(h == 0)
    def _hop2():
        new = jnp.dot(a16_blk, y1_ref[...].astype(jnp.bfloat16),
                      preferred_element_type=jnp.float32)
        y2_ref[rows, :] = new
        acc_ref[rows, :] = acc_ref[rows, :] + a_ref[2] * new

    @pl.when(h == 1)
    def _hop3():
        new = jnp.dot(a16_blk, y2_ref[...].astype(jnp.bfloat16),
                      preferred_element_type=jnp.float32)
        out_ref[rows, :] = acc_ref[rows, :] + a_ref[3] * new + b_ref[...]


def kernel(node_indices, adj_norm, embedding, W, b, alpha):
    a = jax.nn.softmax(alpha.astype(jnp.float32), axis=0)
    x0 = jnp.take(embedding, node_indices, axis=0)
    wt = W.T
    b2 = b.reshape(1, C)

    y0, y1, a16 = pl.pallas_call(
        _sweep1_body,
        grid=(N // BM1,),
        in_specs=[
            pl.BlockSpec((N, D), lambda i: (0, 0)),
            pl.BlockSpec((D, C), lambda i: (0, 0)),
            pl.BlockSpec((BM1, N), lambda i: (i, 0)),
        ],
        out_specs=[
            pl.BlockSpec((N, C), lambda i: (0, 0)),
            pl.BlockSpec((N, C), lambda i: (0, 0)),
            pl.BlockSpec((BM1, N), lambda i: (i, 0)),
        ],
        out_shape=[
            jax.ShapeDtypeStruct((N, C), jnp.float32),
            jax.ShapeDtypeStruct((N, C), jnp.float32),
            jax.ShapeDtypeStruct((N, N), jnp.bfloat16),
        ],
        compiler_params=pltpu.CompilerParams(
            dimension_semantics=("arbitrary",),
        ),
    )(x0, wt, adj_norm)

    out = pl.pallas_call(
        _sweep23_body,
        grid=(2, N // BM2),
        in_specs=[
            pl.BlockSpec(memory_space=pltpu.SMEM),
            pl.BlockSpec((1, C), lambda h, i: (0, 0)),
            pl.BlockSpec((N, C), lambda h, i: (0, 0)),
            pl.BlockSpec((N, C), lambda h, i: (0, 0)),
            pl.BlockSpec((BM2, N), lambda h, i: (i, 0)),
        ],
        out_specs=pl.BlockSpec((N, C), lambda h, i: (0, 0)),
        out_shape=jax.ShapeDtypeStruct((N, C), jnp.float32),
        scratch_shapes=[
            pltpu.VMEM((N, C), jnp.float32),
            pltpu.VMEM((N, C), jnp.float32),
        ],
        compiler_params=pltpu.CompilerParams(
            dimension_semantics=("arbitrary", "arbitrary"),
        ),
    )(a, b2, y0, y1, a16)
    return out
